# Initial kernel scaffold; baseline (speedup 1.0000x reference)
import jax
import jax.numpy as jnp
from jax.experimental import pallas as pl


def kernel(continuous, binary, controller, action, jumps_left, character, l_cancel, hurtbox_state, ground, last_attack_landed, W_action, W_jumps, W_character, W_l_cancel, W_hurtbox, W_ground, W_last_attack):
    raise NotImplementedError("write your pallas kernel here")


# speedup vs baseline: 4.3174x; 4.3174x over previous
import functools

import jax
import jax.numpy as jnp
from jax import lax
from jax.experimental import pallas as pl
from jax.experimental.pallas import tpu as pltpu
from jax.experimental.pallas import tpu_sc as plsc

NC = 2
NS = 16
L = 16
NW = NC * NS

B = 16384
BPW = B // NW
NCHUNK = BPW // L

CONT_PARTS = ((13, 0), (3, 13), (13, 16))
EMB_PARTS = ((32, 29), (4, 61), (8, 65), (2, 73),
             (2, 75), (4, 77), (8, 81))
D_OUT = 89

_TABLE_SIZES = (400 * 32, 8 * 4, 33 * 8, 3 * 2, 3 * 2, 32 * 4, 64 * 8)

_mesh = plsc.VectorSubcoreMesh(
    core_axis_name="c", subcore_axis_name="s", num_cores=NC, num_subcores=NS)


@functools.partial(
    pl.kernel,
    mesh=_mesh,
    compiler_params=pltpu.CompilerParams(needs_layout_passes=False),
    out_type=jax.ShapeDtypeStruct((B * D_OUT,), jnp.float32),
    scratch_types=(
        [pltpu.VMEM((BPW * 13,), jnp.float32),
         pltpu.VMEM((BPW * 3,), jnp.float32),
         pltpu.VMEM((BPW * 13,), jnp.float32)]
        + [pltpu.VMEM((BPW,), jnp.int32) for _ in range(7)]
        + [pltpu.VMEM((n,), jnp.float32) for n in _TABLE_SIZES]
        + [pltpu.VMEM((BPW * D_OUT,), jnp.float32)]
    ),
)
def _encode(cont_h, bin_h, ctrl_h,
            act_h, jmp_h, chr_h, lc_h, hb_h, gnd_h, la_h,
            wa_h, wj_h, wc_h, wl_h, wh_h, wg_h, wla_h,
            out_h,
            cont_v, bin_v, ctrl_v,
            act_v, jmp_v, chr_v, lc_v, hb_v, gnd_v, la_v,
            wa_v, wj_v, wc_v, wl_v, wh_v, wg_v, wla_v,
            out_v):
  wid = lax.axis_index("s") * NC + lax.axis_index("c")
  base = wid * BPW

  for h, v, w in ((cont_h, cont_v, 13), (bin_h, bin_v, 3), (ctrl_h, ctrl_v, 13)):
    pltpu.sync_copy(h.at[pl.ds(base * w, BPW * w)], v)
  idx_refs = (act_v, jmp_v, chr_v, lc_v, hb_v, gnd_v, la_v)
  for h, v in zip((act_h, jmp_h, chr_h, lc_h, hb_h, gnd_h, la_h), idx_refs):
    pltpu.sync_copy(h.at[pl.ds(base, BPW)], v)
  tbl_refs = (wa_v, wj_v, wc_v, wl_v, wh_v, wg_v, wla_v)
  for h, v in zip((wa_h, wj_h, wc_h, wl_h, wh_h, wg_h, wla_h), tbl_refs):
    pltpu.sync_copy(h, v)

  lane = lax.iota(jnp.int32, L)

  def chunk(k, carry):
    rows = lane + k * L
    out_base = rows * D_OUT
    for src, (w, off) in zip((cont_v, bin_v, ctrl_v), CONT_PARTS):
      src_base = rows * w
      for c in range(w):
        v = plsc.load_gather(src, [src_base + c])
        plsc.store_scatter(out_v, [out_base + (off + c)], v)
    for iv, tv, (w, off) in zip(idx_refs, tbl_refs, EMB_PARTS):
      tbl_base = iv[pl.ds(k * L, L)] * w
      for c in range(w):
        v = plsc.load_gather(tv, [tbl_base + c])
        plsc.store_scatter(out_v, [out_base + (off + c)], v)
    return carry

  lax.fori_loop(0, NCHUNK, chunk, 0)
  pltpu.sync_copy(out_v, out_h.at[pl.ds(base * D_OUT, BPW * D_OUT)])


def kernel(continuous, binary, controller, action, jumps_left, character,
           l_cancel, hurtbox_state, ground, last_attack_landed,
           W_action, W_jumps, W_character, W_l_cancel, W_hurtbox, W_ground,
           W_last_attack):
  to_i32 = lambda x: x.astype(jnp.int32)
  flat = lambda x: x.reshape(-1)
  out = _encode(flat(continuous), flat(binary), flat(controller),
                to_i32(action), to_i32(jumps_left), to_i32(character),
                to_i32(l_cancel), to_i32(hurtbox_state), to_i32(ground),
                to_i32(last_attack_landed),
                flat(W_action), flat(W_jumps), flat(W_character),
                flat(W_l_cancel), flat(W_hurtbox), flat(W_ground),
                flat(W_last_attack))
  return out.reshape(B, D_OUT)


# speedup vs baseline: 5.0949x; 1.1801x over previous
import functools

import jax
import jax.numpy as jnp
from jax import lax
from jax.experimental import pallas as pl
from jax.experimental.pallas import tpu as pltpu
from jax.experimental.pallas import tpu_sc as plsc

NC = 2
NS = 16
L = 16
NW = NC * NS

B = 16384
BPW = B // NW
NCHUNK = BPW // L

CONT_PARTS = ((13, 0), (3, 13), (13, 16))
EMB_PARTS = ((32, 29), (4, 61), (8, 65), (2, 73),
             (2, 75), (4, 77), (8, 81))
D_OUT = 89

_TABLE_SIZES = (400 * 32, 8 * 4, 33 * 8, 3 * 2, 3 * 2, 32 * 4, 64 * 8)

_mesh = plsc.VectorSubcoreMesh(
    core_axis_name="c", subcore_axis_name="s", num_cores=NC, num_subcores=NS)


@functools.partial(
    pl.kernel,
    mesh=_mesh,
    compiler_params=pltpu.CompilerParams(needs_layout_passes=False),
    out_type=jax.ShapeDtypeStruct((B * D_OUT,), jnp.float32),
    scratch_types=(
        [pltpu.VMEM((BPW * 13,), jnp.float32),
         pltpu.VMEM((BPW * 3,), jnp.float32),
         pltpu.VMEM((BPW * 13,), jnp.float32)]
        + [pltpu.VMEM((BPW,), jnp.int32) for _ in range(7)]
        + [pltpu.VMEM((n,), jnp.float32) for n in _TABLE_SIZES]
        + [pltpu.VMEM((BPW * D_OUT,), jnp.float32)]
        + [pltpu.SemaphoreType.DMA]
    ),
)
def _encode(cont_h, bin_h, ctrl_h,
            act_h, jmp_h, chr_h, lc_h, hb_h, gnd_h, la_h,
            wa_h, wj_h, wc_h, wl_h, wh_h, wg_h, wla_h,
            out_h,
            cont_v, bin_v, ctrl_v,
            act_v, jmp_v, chr_v, lc_v, hb_v, gnd_v, la_v,
            wa_v, wj_v, wc_v, wl_v, wh_v, wg_v, wla_v,
            out_v, dma_sem):
  wid = lax.axis_index("s") * NC + lax.axis_index("c")
  base = wid * BPW

  idx_refs = (act_v, jmp_v, chr_v, lc_v, hb_v, gnd_v, la_v)
  tbl_refs = (wa_v, wj_v, wc_v, wl_v, wh_v, wg_v, wla_v)
  copies = []
  for h, v, w in ((cont_h, cont_v, 13), (bin_h, bin_v, 3), (ctrl_h, ctrl_v, 13)):
    copies.append(pltpu.async_copy(h.at[pl.ds(base * w, BPW * w)], v, dma_sem))
  for h, v in zip((act_h, jmp_h, chr_h, lc_h, hb_h, gnd_h, la_h), idx_refs):
    copies.append(pltpu.async_copy(h.at[pl.ds(base, BPW)], v, dma_sem))
  for h, v in zip((wa_h, wj_h, wc_h, wl_h, wh_h, wg_h, wla_h), tbl_refs):
    copies.append(pltpu.async_copy(h, v, dma_sem))
  for c in copies:
    c.wait()

  lane = lax.iota(jnp.int32, L)
  GRP = 8

  def chunk(k, carry):
    rows = lane + k * L
    out_base = rows * D_OUT
    jobs = []
    for src, (w, off) in zip((cont_v, bin_v, ctrl_v), CONT_PARTS):
      src_base = rows * w
      for c in range(w):
        jobs.append((src, src_base + c, off + c))
    for iv, tv, (w, off) in zip(idx_refs, tbl_refs, EMB_PARTS):
      tbl_base = iv[pl.ds(k * L, L)] * w
      for c in range(w):
        jobs.append((tv, tbl_base + c, off + c))
    for g in range(0, len(jobs), GRP):
      grp = jobs[g:g + GRP]
      vals = [plsc.load_gather(src, [si]) for src, si, _ in grp]
      for (_, _, oc), v in zip(grp, vals):
        plsc.store_scatter(out_v, [out_base + oc], v)
    return carry

  lax.fori_loop(0, NCHUNK, chunk, 0)
  pltpu.sync_copy(out_v, out_h.at[pl.ds(base * D_OUT, BPW * D_OUT)])


def kernel(continuous, binary, controller, action, jumps_left, character,
           l_cancel, hurtbox_state, ground, last_attack_landed,
           W_action, W_jumps, W_character, W_l_cancel, W_hurtbox, W_ground,
           W_last_attack):
  to_i32 = lambda x: x.astype(jnp.int32)
  flat = lambda x: x.reshape(-1)
  out = _encode(flat(continuous), flat(binary), flat(controller),
                to_i32(action), to_i32(jumps_left), to_i32(character),
                to_i32(l_cancel), to_i32(hurtbox_state), to_i32(ground),
                to_i32(last_attack_landed),
                flat(W_action), flat(W_jumps), flat(W_character),
                flat(W_l_cancel), flat(W_hurtbox), flat(W_ground),
                flat(W_last_attack))
  return out.reshape(B, D_OUT)
